# Initial kernel scaffold; baseline (speedup 1.0000x reference)
import jax
import jax.numpy as jnp
from jax.experimental import pallas as pl


def kernel(X):
    raise NotImplementedError("write your pallas kernel here")


# speedup vs baseline: 1.7550x; 1.7550x over previous
import jax
import jax.numpy as jnp
import numpy as np
from jax.experimental import pallas as pl
from jax.experimental.pallas import tpu as pltpu

_B = 16


def _pool_kernel(x_ref, s_ref, o_ref):
    R = x_ref.shape[0]
    W = x_ref.shape[2]
    h = jnp.maximum(x_ref[:, 0, :], x_ref[:, 1, :])
    c = jnp.dot(h, s_ref[...], preferred_element_type=jnp.float32)
    o_ref[...] = jnp.maximum(c[:, : W // 2], c[:, 128: 128 + W // 2])


def _selector(W):
    S = np.zeros((W, 256), np.float32)
    w = np.arange(W // 2)
    S[2 * w, w] = 1.0
    S[2 * w + 1, 128 + w] = 1.0
    return jnp.asarray(S)


def kernel(X):
    N, C, H, W = X.shape
    NC = N * C
    R = _B * (H // 2)
    x = X.reshape(NC * (H // 2), 2, W)
    grid = ((NC * (H // 2)) // R,)
    out = pl.pallas_call(
        _pool_kernel,
        out_shape=jax.ShapeDtypeStruct((NC * (H // 2), W // 2), X.dtype),
        grid=grid,
        in_specs=[
            pl.BlockSpec((R, 2, W), lambda i: (i, 0, 0)),
            pl.BlockSpec((W, 256), lambda i: (0, 0)),
        ],
        out_specs=pl.BlockSpec((R, W // 2), lambda i: (i, 0)),
        compiler_params=pltpu.CompilerParams(
            dimension_semantics=("arbitrary",),
        ),
        name="maxpool2d_2x2",
    )(x, _selector(W))
    return out.reshape(N, C, H // 2, W // 2)


# speedup vs baseline: 6.1756x; 3.5190x over previous
import jax
import jax.numpy as jnp
import numpy as np
from jax.experimental import pallas as pl
from jax.experimental.pallas import tpu as pltpu

_B = 16


def _pool_kernel(x_ref, s_ref, o_ref, w_ref):
    x = x_ref[...]
    c = jnp.dot(x, s_ref[...], preferred_element_type=jnp.float32)
    w_ref[...] = jnp.maximum(c[:, :128], c[:, 128:])
    h = jnp.maximum(w_ref[0::2, :], w_ref[1::2, :])
    o_ref[...] = h[:, :112]


def _selector(W):
    S = np.zeros((W, 256), np.float32)
    w = np.arange(W // 2)
    S[2 * w, w] = 1.0
    S[2 * w + 1, 128 + w] = 1.0
    return jnp.asarray(S)


def kernel(X):
    N, C, H, W = X.shape
    NC = N * C
    x = X.reshape(NC * H, W)
    grid = (NC // _B,)
    out = pl.pallas_call(
        _pool_kernel,
        out_shape=jax.ShapeDtypeStruct((NC * (H // 2), W // 2), X.dtype),
        grid=grid,
        in_specs=[
            pl.BlockSpec((_B * H, W), lambda i: (i, 0)),
            pl.BlockSpec((W, 256), lambda i: (0, 0)),
        ],
        out_specs=pl.BlockSpec((_B * (H // 2), W // 2), lambda i: (i, 0)),
        scratch_shapes=[pltpu.VMEM((_B * H, 128), jnp.float32)],
        compiler_params=pltpu.CompilerParams(
            dimension_semantics=("arbitrary",),
        ),
        name="maxpool2d_2x2",
    )(x, _selector(W))
    return out.reshape(N, C, H // 2, W // 2)


# speedup vs baseline: 7.4939x; 1.2135x over previous
import jax
import jax.numpy as jnp
import numpy as np
from jax.experimental import pallas as pl
from jax.experimental.pallas import tpu as pltpu

_B = 32


def _pool_kernel(x_ref, s_ref, o_ref, w_ref):
    x = x_ref[...]
    c = jnp.dot(x, s_ref[...], preferred_element_type=jnp.float32)
    w_ref[...] = jnp.maximum(c[:, :128], c[:, 128:])
    h = jnp.maximum(w_ref[0::2, :], w_ref[1::2, :])
    o_ref[...] = h[:, :112]


def _selector(W):
    S = np.zeros((W, 256), np.float32)
    w = np.arange(W // 2)
    S[2 * w, w] = 1.0
    S[2 * w + 1, 128 + w] = 1.0
    return jnp.asarray(S)


def kernel(X):
    N, C, H, W = X.shape
    NC = N * C
    x = X.reshape(NC * H, W)
    grid = (NC // _B,)
    out = pl.pallas_call(
        _pool_kernel,
        out_shape=jax.ShapeDtypeStruct((NC * (H // 2), W // 2), X.dtype),
        grid=grid,
        in_specs=[
            pl.BlockSpec((_B * H, W), lambda i: (i, 0)),
            pl.BlockSpec((W, 256), lambda i: (0, 0)),
        ],
        out_specs=pl.BlockSpec((_B * (H // 2), W // 2), lambda i: (i, 0)),
        scratch_shapes=[pltpu.VMEM((_B * H, 128), jnp.float32)],
        compiler_params=pltpu.CompilerParams(
            dimension_semantics=("arbitrary",),
        ),
        name="maxpool2d_2x2",
    )(x, _selector(W))
    return out.reshape(N, C, H // 2, W // 2)


# speedup vs baseline: 7.6978x; 1.0272x over previous
import jax
import jax.numpy as jnp
import numpy as np
from jax.experimental import pallas as pl
from jax.experimental.pallas import tpu as pltpu

_B = 64


def _pool_kernel(x_ref, s_ref, o_ref, w_ref):
    x = x_ref[...]
    c = jnp.dot(x, s_ref[...], preferred_element_type=jnp.float32)
    w_ref[...] = jnp.maximum(c[:, :128], c[:, 128:])
    h = jnp.maximum(w_ref[0::2, :], w_ref[1::2, :])
    o_ref[...] = h[:, :112]


def _selector(W):
    S = np.zeros((W, 256), np.float32)
    w = np.arange(W // 2)
    S[2 * w, w] = 1.0
    S[2 * w + 1, 128 + w] = 1.0
    return jnp.asarray(S)


def kernel(X):
    N, C, H, W = X.shape
    NC = N * C
    x = X.reshape(NC * H, W)
    grid = (NC // _B,)
    out = pl.pallas_call(
        _pool_kernel,
        out_shape=jax.ShapeDtypeStruct((NC * (H // 2), W // 2), X.dtype),
        grid=grid,
        in_specs=[
            pl.BlockSpec((_B * H, W), lambda i: (i, 0)),
            pl.BlockSpec((W, 256), lambda i: (0, 0)),
        ],
        out_specs=pl.BlockSpec((_B * (H // 2), W // 2), lambda i: (i, 0)),
        scratch_shapes=[pltpu.VMEM((_B * H, 128), jnp.float32)],
        compiler_params=pltpu.CompilerParams(
            dimension_semantics=("arbitrary",),
            vmem_limit_bytes=100 * 1024 * 1024,
        ),
        name="maxpool2d_2x2",
    )(x, _selector(W))
    return out.reshape(N, C, H // 2, W // 2)
